# baseline (speedup 1.0000x reference)
import jax
import jax.numpy as jnp
from jax.experimental import pallas as pl

TOK_BLOCK = 256


def _gating_kernel(x_ref, wt_ref, b_ref, out_ref):
    logits = jnp.dot(x_ref[...], wt_ref[...], preferred_element_type=jnp.float32)
    logits = logits + b_ref[...]
    m = jnp.max(logits, axis=-1, keepdims=True)
    e = jnp.exp(logits - m)
    out_ref[...] = e / jnp.sum(e, axis=-1, keepdims=True)


def kernel(x, W, b):
    tokens, dim = x.shape
    experts = W.shape[0]
    wt = W.T
    b2 = b.reshape(1, experts)
    return pl.pallas_call(
        _gating_kernel,
        grid=(tokens // TOK_BLOCK,),
        in_specs=[
            pl.BlockSpec((TOK_BLOCK, dim), lambda i: (i, 0)),
            pl.BlockSpec((dim, experts), lambda i: (0, 0)),
            pl.BlockSpec((1, experts), lambda i: (0, 0)),
        ],
        out_specs=pl.BlockSpec((TOK_BLOCK, experts), lambda i: (i, 0)),
        out_shape=jax.ShapeDtypeStruct((tokens, experts), jnp.float32),
    )(x, wt, b2)


# speedup vs baseline: 1.2117x; 1.2117x over previous
import jax
import jax.numpy as jnp
from jax.experimental import pallas as pl

TOK_BLOCK = 512


def _gating_kernel(x_ref, wt_ref, b_ref, out_ref):
    logits = jnp.dot(x_ref[...], wt_ref[...], preferred_element_type=jnp.float32)
    logits = logits + b_ref[...]
    m = jnp.max(logits, axis=-1, keepdims=True)
    e = jnp.exp(logits - m)
    out_ref[...] = e / jnp.sum(e, axis=-1, keepdims=True)


def kernel(x, W, b):
    tokens, dim = x.shape
    experts = W.shape[0]
    wt = W.T
    b2 = b.reshape(1, experts)
    return pl.pallas_call(
        _gating_kernel,
        grid=(tokens // TOK_BLOCK,),
        in_specs=[
            pl.BlockSpec((TOK_BLOCK, dim), lambda i: (i, 0)),
            pl.BlockSpec((dim, experts), lambda i: (0, 0)),
            pl.BlockSpec((1, experts), lambda i: (0, 0)),
        ],
        out_specs=pl.BlockSpec((TOK_BLOCK, experts), lambda i: (i, 0)),
        out_shape=jax.ShapeDtypeStruct((tokens, experts), jnp.float32),
    )(x, wt, b2)


# speedup vs baseline: 1.2217x; 1.0083x over previous
import jax
import jax.numpy as jnp
from jax.experimental import pallas as pl

TOK_BLOCK = 1024


def _gating_kernel(x_ref, wt_ref, b_ref, out_ref):
    logits = jnp.dot(x_ref[...], wt_ref[...], preferred_element_type=jnp.float32)
    logits = logits + b_ref[...]
    m = jnp.max(logits, axis=-1, keepdims=True)
    e = jnp.exp(logits - m)
    out_ref[...] = e / jnp.sum(e, axis=-1, keepdims=True)


def kernel(x, W, b):
    tokens, dim = x.shape
    experts = W.shape[0]
    wt = W.T
    b2 = b.reshape(1, experts)
    return pl.pallas_call(
        _gating_kernel,
        grid=(tokens // TOK_BLOCK,),
        in_specs=[
            pl.BlockSpec((TOK_BLOCK, dim), lambda i: (i, 0)),
            pl.BlockSpec((dim, experts), lambda i: (0, 0)),
            pl.BlockSpec((1, experts), lambda i: (0, 0)),
        ],
        out_specs=pl.BlockSpec((TOK_BLOCK, experts), lambda i: (i, 0)),
        out_shape=jax.ShapeDtypeStruct((tokens, experts), jnp.float32),
    )(x, wt, b2)


# speedup vs baseline: 1.2228x; 1.0009x over previous
import jax
import jax.numpy as jnp
from jax.experimental import pallas as pl

HALF = 512


def _softmax_rows(logits):
    m = jnp.max(logits, axis=-1, keepdims=True)
    e = jnp.exp(logits - m)
    return e / jnp.sum(e, axis=-1, keepdims=True)


def _gating_kernel(x0_ref, x1_ref, wt_ref, b_ref, out_ref):
    wt = wt_ref[...]
    b = b_ref[...]
    l0 = jnp.dot(x0_ref[...], wt, preferred_element_type=jnp.float32) + b
    out_ref[0:HALF, :] = _softmax_rows(l0)
    l1 = jnp.dot(x1_ref[...], wt, preferred_element_type=jnp.float32) + b
    out_ref[HALF : 2 * HALF, :] = _softmax_rows(l1)


def kernel(x, W, b):
    tokens, dim = x.shape
    experts = W.shape[0]
    wt = W.T
    b2 = b.reshape(1, experts)
    return pl.pallas_call(
        _gating_kernel,
        grid=(tokens // (2 * HALF),),
        in_specs=[
            pl.BlockSpec((HALF, dim), lambda i: (2 * i, 0)),
            pl.BlockSpec((HALF, dim), lambda i: (2 * i + 1, 0)),
            pl.BlockSpec((dim, experts), lambda i: (0, 0)),
            pl.BlockSpec((1, experts), lambda i: (0, 0)),
        ],
        out_specs=pl.BlockSpec((2 * HALF, experts), lambda i: (i, 0)),
        out_shape=jax.ShapeDtypeStruct((tokens, experts), jnp.float32),
    )(x, x, wt, b2)
